# baseline (speedup 1.0000x reference)
import functools

import jax
import jax.numpy as jnp
from jax import lax
from jax.experimental import pallas as pl
from jax.experimental.pallas import tpu as pltpu
from jax.experimental.pallas import tpu_sc as plsc

_WINDOW = 128


def _sc_gather(emb_table, lin2d, idx, idx16):
    n = idx.shape[1]
    d = emb_table.shape[1]
    mesh = plsc.VectorSubcoreMesh(core_axis_name="c", subcore_axis_name="s")

    @functools.partial(
        pl.kernel,
        out_type=[
            jax.ShapeDtypeStruct((n, d), emb_table.dtype),
            jax.ShapeDtypeStruct((n, 16), lin2d.dtype),
        ],
        mesh=mesh,
        compiler_params=pltpu.CompilerParams(use_tc_tiling_on_sc=False),
    )
    def gather_kernel(emb_hbm, lin_hbm, i_hbm, i16_hbm, emb_out, lin_out):
        def body(i_vmem, i16_vmem, emb_vmem, lin_vmem):
            pltpu.sync_copy(emb_hbm.at[i_vmem.at[0]], emb_vmem)
            pltpu.sync_copy(lin_hbm.at[i16_vmem.at[0]], lin_vmem)

        pltpu.emit_pipeline(
            body,
            grid=(n // _WINDOW,),
            in_specs=[
                pl.BlockSpec((1, _WINDOW), lambda i: (0, i)),
                pl.BlockSpec((1, _WINDOW), lambda i: (0, i)),
            ],
            out_specs=[
                pl.BlockSpec((_WINDOW, d), lambda i: (i, 0)),
                pl.BlockSpec((_WINDOW, 16), lambda i: (i, 0)),
            ],
            core_axis_name=("c", "s"),
            dimension_semantics=(pltpu.PARALLEL,),
        )(i_hbm, i16_hbm, emb_out, lin_out)

    return gather_kernel(emb_table, lin2d, idx, idx16)


def _transpose_body(xt_ref, o_ref):
    xt = xt_ref[...]
    d, vb = xt.shape
    ng = vb // 1024
    x4 = xt.reshape(d, ng, 8, 128)
    outs = []
    for g in range(ng):
        bg = x4[:, g].transpose((1, 0, 2)).reshape(128, 128)
        outs.append(bg.T)
    o_ref[...] = jnp.concatenate(outs, axis=0)


def _tc_relayout(emb_t, block_v=4096):
    d, v = emb_t.shape
    grid = (v + block_v - 1) // block_v
    return pl.pallas_call(
        _transpose_body,
        grid=(grid,),
        in_specs=[pl.BlockSpec((d, block_v), lambda i: (0, i))],
        out_specs=pl.BlockSpec((block_v // 8, 8 * d), lambda i: (i, 0)),
        out_shape=jax.ShapeDtypeStruct((grid * block_v // 8, 8 * d), emb_t.dtype),
    )(emb_t)


def _tc_body(e_ref, lr_ref, xm_ref, bias_ref, w1_ref, b1_ref, w2_ref, b2_ref,
             o_ref):
    e = e_ref[...]
    fd = w1_ref.shape[0]
    f = xm_ref.shape[1]
    d = fd // f
    sel = (
        lax.broadcasted_iota(jnp.int32, (fd, d), 0) % d
        == lax.broadcasted_iota(jnp.int32, (fd, d), 1)
    ).astype(jnp.float32)
    s = jnp.dot(e, sel, preferred_element_type=jnp.float32)
    fm = 0.5 * (
        jnp.sum(s * s, axis=1, keepdims=True)
        - jnp.sum(e * e, axis=1, keepdims=True)
    )
    rep_sel = (
        lax.broadcasted_iota(jnp.int32, (f, fd), 0)
        == lax.broadcasted_iota(jnp.int32, (f, fd), 1) // d
    ).astype(jnp.float32)
    rep = jnp.dot(xm_ref[...], rep_sel, preferred_element_type=jnp.float32)
    jpat = (lax.broadcasted_iota(jnp.int32, (1, fd), 1) % d).astype(jnp.float32)
    mask = (rep == jpat).astype(jnp.float32)
    lin = jnp.sum(lr_ref[...] * mask, axis=1, keepdims=True) + bias_ref[0, 0]
    e16 = e.astype(jnp.bfloat16)
    h = jnp.dot(e16, w1_ref[...], preferred_element_type=jnp.float32)
    h = jnp.maximum(h + b1_ref[...], 0.0).astype(jnp.bfloat16)
    h2 = jnp.dot(h, w2_ref[...], preferred_element_type=jnp.float32)
    h2 = jnp.maximum(h2 + b2_ref[...], 0.0)
    mlp = jnp.sum(h2, axis=1, keepdims=True)
    o_ref[...] = jax.nn.sigmoid(lin + fm + mlp)


def _tc_deepfm(e, lr, xm, bias, w1, b1, w2, b2, block_b=1024):
    b, fd = e.shape
    f = xm.shape[1]
    h1, h2 = w1.shape[1], w2.shape[1]
    return pl.pallas_call(
        _tc_body,
        grid=(b // block_b,),
        in_specs=[
            pl.BlockSpec((block_b, fd), lambda i: (i, 0)),
            pl.BlockSpec((block_b, f * 16), lambda i: (i, 0)),
            pl.BlockSpec((block_b, f), lambda i: (i, 0)),
            pl.BlockSpec((1, 1), lambda i: (0, 0)),
            pl.BlockSpec((fd, h1), lambda i: (0, 0)),
            pl.BlockSpec((1, h1), lambda i: (0, 0)),
            pl.BlockSpec((h1, h2), lambda i: (0, 0)),
            pl.BlockSpec((1, h2), lambda i: (0, 0)),
        ],
        out_specs=pl.BlockSpec((block_b, 1), lambda i: (i, 0)),
        out_shape=jax.ShapeDtypeStruct((b, 1), jnp.float32),
    )(e, lr, xm, bias, w1, b1, w2, b2)


def kernel(x, emb_table, lin_table, bias, W1, b1, W2, b2):
    b, f = x.shape
    v, d = emb_table.shape
    xi = x.astype(jnp.int32)
    idx = xi.reshape(1, b * f)
    idx16 = idx >> 4
    xm = (xi & 15).astype(jnp.float32)
    lin2d = lin_table.reshape(v // 16, 16)
    emb_perm = _tc_relayout(emb_table.T)
    emb_lin = emb_perm.reshape(emb_perm.shape[0] * 8, d)
    kperm = ((idx >> 10) << 10) + ((idx & 127) << 3) + ((idx >> 7) & 7)
    emb_g, lin_g = _sc_gather(emb_lin, lin2d, kperm, idx16)
    _probe = jnp.sum(emb_g, axis=1) + jnp.sum(lin_g, axis=1)
    return jax.nn.sigmoid(jnp.sum(_probe.reshape(b, f), axis=1))
    e = emb_g.reshape(b, f * d)
    lr = lin_g.reshape(b, f * 16)
    out = _tc_deepfm(
        e,
        lr,
        xm,
        bias.reshape(1, 1),
        W1.astype(jnp.bfloat16),
        b1.reshape(1, -1),
        W2.astype(jnp.bfloat16),
        b2.reshape(1, -1),
    )
    return out.reshape(b)


# speedup vs baseline: 1.6369x; 1.6369x over previous
import functools

import jax
import jax.numpy as jnp
from jax import lax
from jax.experimental import pallas as pl
from jax.experimental.pallas import tpu as pltpu
from jax.experimental.pallas import tpu_sc as plsc

_WINDOW = 128


def _sc_gather(emb_table, lin2d, idx, idx16):
    n = idx.shape[1]
    d = emb_table.shape[1]
    mesh = plsc.VectorSubcoreMesh(core_axis_name="c", subcore_axis_name="s")

    @functools.partial(
        pl.kernel,
        out_type=[
            jax.ShapeDtypeStruct((n, d), emb_table.dtype),
            jax.ShapeDtypeStruct((n, 16), lin2d.dtype),
        ],
        mesh=mesh,
        compiler_params=pltpu.CompilerParams(use_tc_tiling_on_sc=False),
    )
    def gather_kernel(emb_hbm, lin_hbm, i_hbm, i16_hbm, emb_out, lin_out):
        def body(i_vmem, i16_vmem, emb_vmem, lin_vmem):
            pltpu.sync_copy(emb_hbm.at[i_vmem.at[0]], emb_vmem)
            pltpu.sync_copy(lin_hbm.at[i16_vmem.at[0]], lin_vmem)

        pltpu.emit_pipeline(
            body,
            grid=(n // _WINDOW,),
            in_specs=[
                pl.BlockSpec((1, _WINDOW), lambda i: (0, i)),
                pl.BlockSpec((1, _WINDOW), lambda i: (0, i)),
            ],
            out_specs=[
                pl.BlockSpec((_WINDOW, d), lambda i: (i, 0)),
                pl.BlockSpec((_WINDOW, 16), lambda i: (i, 0)),
            ],
            core_axis_name=("c", "s"),
            dimension_semantics=(pltpu.PARALLEL,),
        )(i_hbm, i16_hbm, emb_out, lin_out)

    return gather_kernel(emb_table, lin2d, idx, idx16)


def _transpose_body(xt_ref, o_ref):
    xt = xt_ref[...]
    d, vb = xt.shape
    ng = vb // 1024
    x4 = xt.reshape(d, ng, 8, 128)
    outs = []
    for g in range(ng):
        bg = x4[:, g].transpose((1, 0, 2)).reshape(128, 128)
        outs.append(bg.T)
    o_ref[...] = jnp.concatenate(outs, axis=0)


def _tc_relayout(emb_t, block_v=8192):
    d, v = emb_t.shape
    grid = (v + block_v - 1) // block_v
    return pl.pallas_call(
        _transpose_body,
        grid=(grid,),
        in_specs=[pl.BlockSpec((d, block_v), lambda i: (0, i))],
        out_specs=pl.BlockSpec((block_v // 8, 8 * d), lambda i: (i, 0)),
        out_shape=jax.ShapeDtypeStruct((grid * block_v // 8, 8 * d), emb_t.dtype),
    )(emb_t)


def _tc_body(e_ref, lr_ref, xm_ref, bias_ref, w1_ref, b1_ref, w2_ref, b2_ref,
             o_ref):
    e = e_ref[...]
    fd = w1_ref.shape[0]
    f = xm_ref.shape[1]
    d = fd // f
    sel = (
        lax.broadcasted_iota(jnp.int32, (fd, d), 0) % d
        == lax.broadcasted_iota(jnp.int32, (fd, d), 1)
    ).astype(jnp.float32)
    s = jnp.dot(e, sel, preferred_element_type=jnp.float32)
    fm = 0.5 * (
        jnp.sum(s * s, axis=1, keepdims=True)
        - jnp.sum(e * e, axis=1, keepdims=True)
    )
    rep_sel = (
        lax.broadcasted_iota(jnp.int32, (f, fd), 0)
        == lax.broadcasted_iota(jnp.int32, (f, fd), 1) // d
    ).astype(jnp.float32)
    rep = jnp.dot(xm_ref[...], rep_sel, preferred_element_type=jnp.float32)
    jpat = (lax.broadcasted_iota(jnp.int32, (1, fd), 1) % d).astype(jnp.float32)
    mask = (rep == jpat).astype(jnp.float32)
    lin = jnp.sum(lr_ref[...] * mask, axis=1, keepdims=True) + bias_ref[0, 0]
    e16 = e.astype(jnp.bfloat16)
    h = jnp.dot(e16, w1_ref[...], preferred_element_type=jnp.float32)
    h = jnp.maximum(h + b1_ref[...], 0.0).astype(jnp.bfloat16)
    h2 = jnp.dot(h, w2_ref[...], preferred_element_type=jnp.float32)
    h2 = jnp.maximum(h2 + b2_ref[...], 0.0)
    mlp = jnp.sum(h2, axis=1, keepdims=True)
    o_ref[...] = jax.nn.sigmoid(lin + fm + mlp)


def _tc_deepfm(e, lr, xm, bias, w1, b1, w2, b2, block_b=2048):
    b, fd = e.shape
    f = xm.shape[1]
    h1, h2 = w1.shape[1], w2.shape[1]
    return pl.pallas_call(
        _tc_body,
        grid=(b // block_b,),
        in_specs=[
            pl.BlockSpec((block_b, fd), lambda i: (i, 0)),
            pl.BlockSpec((block_b, f * 16), lambda i: (i, 0)),
            pl.BlockSpec((block_b, f), lambda i: (i, 0)),
            pl.BlockSpec((1, 1), lambda i: (0, 0)),
            pl.BlockSpec((fd, h1), lambda i: (0, 0)),
            pl.BlockSpec((1, h1), lambda i: (0, 0)),
            pl.BlockSpec((h1, h2), lambda i: (0, 0)),
            pl.BlockSpec((1, h2), lambda i: (0, 0)),
        ],
        out_specs=pl.BlockSpec((block_b, 1), lambda i: (i, 0)),
        out_shape=jax.ShapeDtypeStruct((b, 1), jnp.float32),
    )(e, lr, xm, bias, w1, b1, w2, b2)


def kernel(x, emb_table, lin_table, bias, W1, b1, W2, b2):
    b, f = x.shape
    v, d = emb_table.shape
    xi = x.astype(jnp.int32)
    idx = xi.reshape(1, b * f)
    idx16 = idx >> 4
    xm = (xi & 15).astype(jnp.float32)
    lin2d = lin_table.reshape(v // 16, 16)
    emb_perm = _tc_relayout(emb_table.T)
    emb_lin = emb_perm.reshape(emb_perm.shape[0] * 8, d)
    kperm = ((idx >> 10) << 10) + ((idx & 127) << 3) + ((idx >> 7) & 7)
    emb_g, lin_g = _sc_gather(emb_lin, lin2d, kperm, idx16)
    e = emb_g.reshape(b, f * d)
    lr = lin_g.reshape(b, f * 16)
    out = _tc_deepfm(
        e,
        lr,
        xm,
        bias.reshape(1, 1),
        W1.astype(jnp.bfloat16),
        b1.reshape(1, -1),
        W2.astype(jnp.bfloat16),
        b2.reshape(1, -1),
    )
    return out.reshape(b)


# speedup vs baseline: 1.8287x; 1.1172x over previous
import functools

import jax
import jax.numpy as jnp
from jax import lax
from jax.experimental import pallas as pl
from jax.experimental.pallas import tpu as pltpu
from jax.experimental.pallas import tpu_sc as plsc

_WINDOW = 128


def _sc_gather(emb_table, lin2d, idx, idx16):
    n = idx.shape[1]
    d = emb_table.shape[1]
    mesh = plsc.VectorSubcoreMesh(core_axis_name="c", subcore_axis_name="s")

    @functools.partial(
        pl.kernel,
        out_type=[
            jax.ShapeDtypeStruct((n, d), emb_table.dtype),
            jax.ShapeDtypeStruct((n, 16), lin2d.dtype),
        ],
        mesh=mesh,
        compiler_params=pltpu.CompilerParams(use_tc_tiling_on_sc=False),
    )
    def gather_kernel(emb_hbm, lin_hbm, i_hbm, i16_hbm, emb_out, lin_out):
        def body(i_vmem, i16_vmem, emb_vmem, lin_vmem):
            pltpu.sync_copy(emb_hbm.at[i_vmem.at[0]], emb_vmem)
            pltpu.sync_copy(lin_hbm.at[i16_vmem.at[0]], lin_vmem)

        pltpu.emit_pipeline(
            body,
            grid=(n // _WINDOW,),
            in_specs=[
                pl.BlockSpec((1, _WINDOW), lambda i: (0, i)),
                pl.BlockSpec((1, _WINDOW), lambda i: (0, i)),
            ],
            out_specs=[
                pl.BlockSpec((_WINDOW, d), lambda i: (i, 0)),
                pl.BlockSpec((_WINDOW, 16), lambda i: (i, 0)),
            ],
            core_axis_name=("c", "s"),
            dimension_semantics=(pltpu.PARALLEL,),
        )(i_hbm, i16_hbm, emb_out, lin_out)

    return gather_kernel(emb_table, lin2d, idx, idx16)


def _transpose_body(xt_ref, o_ref):
    xt = xt_ref[...]
    d, vb = xt.shape
    ng = vb // 1024
    x4 = xt.reshape(d, ng, 8, 128)
    outs = []
    for g in range(ng):
        bg = x4[:, g].transpose((1, 0, 2)).reshape(128, 128)
        outs.append(bg.T)
    o_ref[...] = jnp.concatenate(outs, axis=0)


def _tc_relayout(emb_t, block_v=16384):
    d, v = emb_t.shape
    grid = (v + block_v - 1) // block_v
    return pl.pallas_call(
        _transpose_body,
        grid=(grid,),
        in_specs=[pl.BlockSpec((d, block_v), lambda i: (0, i))],
        out_specs=pl.BlockSpec((block_v // 8, 8 * d), lambda i: (i, 0)),
        out_shape=jax.ShapeDtypeStruct((grid * block_v // 8, 8 * d), emb_t.dtype),
    )(emb_t)


def _tc_body(e_ref, lr_ref, xm_ref, bias_ref, w1_ref, b1_ref, w2_ref, b2_ref,
             o_ref):
    e = e_ref[...]
    fd = w1_ref.shape[0]
    f = xm_ref.shape[1]
    d = fd // f
    sel = (
        lax.broadcasted_iota(jnp.int32, (fd, d), 0) % d
        == lax.broadcasted_iota(jnp.int32, (fd, d), 1)
    ).astype(jnp.float32)
    s = jnp.dot(e, sel, preferred_element_type=jnp.float32)
    fm = 0.5 * (
        jnp.sum(s * s, axis=1, keepdims=True)
        - jnp.sum(e * e, axis=1, keepdims=True)
    )
    rep_sel = (
        lax.broadcasted_iota(jnp.int32, (f, fd), 0)
        == lax.broadcasted_iota(jnp.int32, (f, fd), 1) // d
    ).astype(jnp.float32)
    rep = jnp.dot(xm_ref[...], rep_sel, preferred_element_type=jnp.float32)
    jpat = (lax.broadcasted_iota(jnp.int32, (1, fd), 1) % d).astype(jnp.float32)
    mask = (rep == jpat).astype(jnp.float32)
    lin = jnp.sum(lr_ref[...] * mask, axis=1, keepdims=True) + bias_ref[0, 0]
    e16 = e.astype(jnp.bfloat16)
    h = jnp.dot(e16, w1_ref[...], preferred_element_type=jnp.float32)
    h = jnp.maximum(h + b1_ref[...], 0.0).astype(jnp.bfloat16)
    h2 = jnp.dot(h, w2_ref[...], preferred_element_type=jnp.float32)
    h2 = jnp.maximum(h2 + b2_ref[...], 0.0)
    mlp = jnp.sum(h2, axis=1, keepdims=True)
    o_ref[...] = jax.nn.sigmoid(lin + fm + mlp)


def _tc_deepfm(e, lr, xm, bias, w1, b1, w2, b2, block_b=4096):
    b, fd = e.shape
    f = xm.shape[1]
    h1, h2 = w1.shape[1], w2.shape[1]
    return pl.pallas_call(
        _tc_body,
        grid=(b // block_b,),
        in_specs=[
            pl.BlockSpec((block_b, fd), lambda i: (i, 0)),
            pl.BlockSpec((block_b, f * 16), lambda i: (i, 0)),
            pl.BlockSpec((block_b, f), lambda i: (i, 0)),
            pl.BlockSpec((1, 1), lambda i: (0, 0)),
            pl.BlockSpec((fd, h1), lambda i: (0, 0)),
            pl.BlockSpec((1, h1), lambda i: (0, 0)),
            pl.BlockSpec((h1, h2), lambda i: (0, 0)),
            pl.BlockSpec((1, h2), lambda i: (0, 0)),
        ],
        out_specs=pl.BlockSpec((block_b, 1), lambda i: (i, 0)),
        out_shape=jax.ShapeDtypeStruct((b, 1), jnp.float32),
    )(e, lr, xm, bias, w1, b1, w2, b2)


def kernel(x, emb_table, lin_table, bias, W1, b1, W2, b2):
    b, f = x.shape
    v, d = emb_table.shape
    xi = x.astype(jnp.int32)
    idx = xi.reshape(1, b * f)
    idx16 = idx >> 4
    xm = (xi & 15).astype(jnp.float32)
    lin2d = lin_table.reshape(v // 16, 16)
    emb_perm = _tc_relayout(emb_table.T)
    emb_lin = emb_perm.reshape(emb_perm.shape[0] * 8, d)
    kperm = ((idx >> 10) << 10) + ((idx & 127) << 3) + ((idx >> 7) & 7)
    emb_g, lin_g = _sc_gather(emb_lin, lin2d, kperm, idx16)
    e = emb_g.reshape(b, f * d)
    lr = lin_g.reshape(b, f * 16)
    out = _tc_deepfm(
        e,
        lr,
        xm,
        bias.reshape(1, 1),
        W1.astype(jnp.bfloat16),
        b1.reshape(1, -1),
        W2.astype(jnp.bfloat16),
        b2.reshape(1, -1),
    )
    return out.reshape(b)


# speedup vs baseline: 1.9767x; 1.0809x over previous
import functools

import jax
import jax.numpy as jnp
from jax import lax
from jax.experimental import pallas as pl
from jax.experimental.pallas import tpu as pltpu
from jax.experimental.pallas import tpu_sc as plsc

_WINDOW = 128


def _sc_gather(emb_table, lin2d, idx, idx16):
    n = idx.shape[1]
    d = emb_table.shape[1]
    mesh = plsc.VectorSubcoreMesh(core_axis_name="c", subcore_axis_name="s")

    @functools.partial(
        pl.kernel,
        out_type=[
            jax.ShapeDtypeStruct((n, d), emb_table.dtype),
            jax.ShapeDtypeStruct((n, 16), lin2d.dtype),
        ],
        mesh=mesh,
        compiler_params=pltpu.CompilerParams(use_tc_tiling_on_sc=False),
    )
    def gather_kernel(emb_hbm, lin_hbm, i_hbm, i16_hbm, emb_out, lin_out):
        def body(i_vmem, i16_vmem, emb_vmem, lin_vmem):
            pltpu.sync_copy(emb_hbm.at[i_vmem.at[0]], emb_vmem)
            pltpu.sync_copy(lin_hbm.at[i16_vmem.at[0]], lin_vmem)

        pltpu.emit_pipeline(
            body,
            grid=(n // _WINDOW,),
            in_specs=[
                pl.BlockSpec((1, _WINDOW), lambda i: (0, i)),
                pl.BlockSpec((1, _WINDOW), lambda i: (0, i)),
            ],
            out_specs=[
                pl.BlockSpec((_WINDOW, d), lambda i: (i, 0)),
                pl.BlockSpec((_WINDOW, 16), lambda i: (i, 0)),
            ],
            core_axis_name=("c", "s"),
            dimension_semantics=(pltpu.PARALLEL,),
        )(i_hbm, i16_hbm, emb_out, lin_out)

    return gather_kernel(emb_table, lin2d, idx, idx16)


def _transpose_body(xt_ref, o_ref):
    xt = xt_ref[...]
    d, vb = xt.shape
    ng = vb // 1024
    x4 = xt.reshape(d, ng, 8, 128)
    outs = []
    for g in range(ng):
        bg = x4[:, g].transpose((1, 0, 2)).reshape(128, 128)
        outs.append(bg.T)
    o_ref[...] = jnp.concatenate(outs, axis=0)


def _tc_relayout(emb_t, block_v=32768):
    d, v = emb_t.shape
    grid = (v + block_v - 1) // block_v
    return pl.pallas_call(
        _transpose_body,
        grid=(grid,),
        in_specs=[pl.BlockSpec((d, block_v), lambda i: (0, i))],
        out_specs=pl.BlockSpec((block_v // 8, 8 * d), lambda i: (i, 0)),
        out_shape=jax.ShapeDtypeStruct((grid * block_v // 8, 8 * d), emb_t.dtype),
    )(emb_t)


def _tc_body(e_ref, lr_ref, xm_ref, bias_ref, w1_ref, b1_ref, w2_ref, b2_ref,
             o_ref):
    e = e_ref[...]
    fd = w1_ref.shape[0]
    f = xm_ref.shape[1]
    d = fd // f
    sel = (
        lax.broadcasted_iota(jnp.int32, (fd, d), 0) % d
        == lax.broadcasted_iota(jnp.int32, (fd, d), 1)
    ).astype(jnp.float32)
    s = jnp.dot(e, sel, preferred_element_type=jnp.float32)
    fm = 0.5 * (
        jnp.sum(s * s, axis=1, keepdims=True)
        - jnp.sum(e * e, axis=1, keepdims=True)
    )
    rep_sel = (
        lax.broadcasted_iota(jnp.int32, (f, fd), 0)
        == lax.broadcasted_iota(jnp.int32, (f, fd), 1) // d
    ).astype(jnp.float32)
    rep = jnp.dot(xm_ref[...], rep_sel, preferred_element_type=jnp.float32)
    jpat = (lax.broadcasted_iota(jnp.int32, (1, fd), 1) % d).astype(jnp.float32)
    mask = (rep == jpat).astype(jnp.float32)
    lin = jnp.sum(lr_ref[...] * mask, axis=1, keepdims=True) + bias_ref[0, 0]
    e16 = e.astype(jnp.bfloat16)
    h = jnp.dot(e16, w1_ref[...], preferred_element_type=jnp.float32)
    h = jnp.maximum(h + b1_ref[...], 0.0).astype(jnp.bfloat16)
    h2 = jnp.dot(h, w2_ref[...], preferred_element_type=jnp.float32)
    h2 = jnp.maximum(h2 + b2_ref[...], 0.0)
    mlp = jnp.sum(h2, axis=1, keepdims=True)
    o_ref[...] = jax.nn.sigmoid(lin + fm + mlp)


def _tc_deepfm(e, lr, xm, bias, w1, b1, w2, b2, block_b=4096):
    b, fd = e.shape
    f = xm.shape[1]
    h1, h2 = w1.shape[1], w2.shape[1]
    return pl.pallas_call(
        _tc_body,
        grid=(b // block_b,),
        in_specs=[
            pl.BlockSpec((block_b, fd), lambda i: (i, 0)),
            pl.BlockSpec((block_b, f * 16), lambda i: (i, 0)),
            pl.BlockSpec((block_b, f), lambda i: (i, 0)),
            pl.BlockSpec((1, 1), lambda i: (0, 0)),
            pl.BlockSpec((fd, h1), lambda i: (0, 0)),
            pl.BlockSpec((1, h1), lambda i: (0, 0)),
            pl.BlockSpec((h1, h2), lambda i: (0, 0)),
            pl.BlockSpec((1, h2), lambda i: (0, 0)),
        ],
        out_specs=pl.BlockSpec((block_b, 1), lambda i: (i, 0)),
        out_shape=jax.ShapeDtypeStruct((b, 1), jnp.float32),
    )(e, lr, xm, bias, w1, b1, w2, b2)


def kernel(x, emb_table, lin_table, bias, W1, b1, W2, b2):
    b, f = x.shape
    v, d = emb_table.shape
    xi = x.astype(jnp.int32)
    idx = xi.reshape(1, b * f)
    idx16 = idx >> 4
    xm = (xi & 15).astype(jnp.float32)
    lin2d = lin_table.reshape(v // 16, 16)
    emb_perm = _tc_relayout(emb_table.T)
    emb_lin = emb_perm.reshape(emb_perm.shape[0] * 8, d)
    kperm = ((idx >> 10) << 10) + ((idx & 127) << 3) + ((idx >> 7) & 7)
    emb_g, lin_g = _sc_gather(emb_lin, lin2d, kperm, idx16)
    e = emb_g.reshape(b, f * d)
    lr = lin_g.reshape(b, f * 16)
    out = _tc_deepfm(
        e,
        lr,
        xm,
        bias.reshape(1, 1),
        W1.astype(jnp.bfloat16),
        b1.reshape(1, -1),
        W2.astype(jnp.bfloat16),
        b2.reshape(1, -1),
    )
    return out.reshape(b)


# speedup vs baseline: 2.0458x; 1.0349x over previous
import functools

import jax
import jax.numpy as jnp
from jax import lax
from jax.experimental import pallas as pl
from jax.experimental.pallas import tpu as pltpu
from jax.experimental.pallas import tpu_sc as plsc

_WINDOW = 128


def _sc_gather(emb_table, lin2d, idx, idx16):
    n = idx.shape[1]
    d = emb_table.shape[1]
    mesh = plsc.VectorSubcoreMesh(core_axis_name="c", subcore_axis_name="s")

    @functools.partial(
        pl.kernel,
        out_type=[
            jax.ShapeDtypeStruct((n, d), emb_table.dtype),
            jax.ShapeDtypeStruct((n, 16), lin2d.dtype),
        ],
        mesh=mesh,
        compiler_params=pltpu.CompilerParams(use_tc_tiling_on_sc=False),
    )
    def gather_kernel(emb_hbm, lin_hbm, i_hbm, i16_hbm, emb_out, lin_out):
        def body(i_vmem, i16_vmem, emb_vmem, lin_vmem):
            pltpu.sync_copy(emb_hbm.at[i_vmem.at[0]], emb_vmem)
            pltpu.sync_copy(lin_hbm.at[i16_vmem.at[0]], lin_vmem)

        pltpu.emit_pipeline(
            body,
            grid=(n // _WINDOW,),
            in_specs=[
                pl.BlockSpec((1, _WINDOW), lambda i: (0, i)),
                pl.BlockSpec((1, _WINDOW), lambda i: (0, i)),
            ],
            out_specs=[
                pl.BlockSpec((_WINDOW, d), lambda i: (i, 0)),
                pl.BlockSpec((_WINDOW, 16), lambda i: (i, 0)),
            ],
            core_axis_name=("c", "s"),
            dimension_semantics=(pltpu.PARALLEL,),
        )(i_hbm, i16_hbm, emb_out, lin_out)

    return gather_kernel(emb_table, lin2d, idx, idx16)


def _transpose_body(xt_ref, o_ref):
    xt = xt_ref[...]
    d, vb = xt.shape
    ng = vb // 1024
    x4 = xt.reshape(d, ng, 8, 128)
    outs = []
    for g in range(ng):
        bg = x4[:, g].transpose((1, 0, 2)).reshape(128, 128)
        outs.append(bg.T)
    o_ref[...] = jnp.concatenate(outs, axis=0)


def _tc_relayout(emb_t, block_v=65536):
    d, v = emb_t.shape
    grid = (v + block_v - 1) // block_v
    return pl.pallas_call(
        _transpose_body,
        grid=(grid,),
        in_specs=[pl.BlockSpec((d, block_v), lambda i: (0, i))],
        out_specs=pl.BlockSpec((block_v // 8, 8 * d), lambda i: (i, 0)),
        out_shape=jax.ShapeDtypeStruct((grid * block_v // 8, 8 * d), emb_t.dtype),
    )(emb_t)


def _tc_body(e_ref, lr_ref, xm_ref, bias_ref, w1_ref, b1_ref, w2_ref, b2_ref,
             o_ref):
    e = e_ref[...]
    fd = w1_ref.shape[0]
    f = xm_ref.shape[1]
    d = fd // f
    sel = (
        lax.broadcasted_iota(jnp.int32, (fd, d), 0) % d
        == lax.broadcasted_iota(jnp.int32, (fd, d), 1)
    ).astype(jnp.float32)
    s = jnp.dot(e, sel, preferred_element_type=jnp.float32)
    fm = 0.5 * (
        jnp.sum(s * s, axis=1, keepdims=True)
        - jnp.sum(e * e, axis=1, keepdims=True)
    )
    rep_sel = (
        lax.broadcasted_iota(jnp.int32, (f, fd), 0)
        == lax.broadcasted_iota(jnp.int32, (f, fd), 1) // d
    ).astype(jnp.float32)
    rep = jnp.dot(xm_ref[...], rep_sel, preferred_element_type=jnp.float32)
    jpat = (lax.broadcasted_iota(jnp.int32, (1, fd), 1) % d).astype(jnp.float32)
    mask = (rep == jpat).astype(jnp.float32)
    lin = jnp.sum(lr_ref[...] * mask, axis=1, keepdims=True) + bias_ref[0, 0]
    e16 = e.astype(jnp.bfloat16)
    h = jnp.dot(e16, w1_ref[...], preferred_element_type=jnp.float32)
    h = jnp.maximum(h + b1_ref[...], 0.0).astype(jnp.bfloat16)
    h2 = jnp.dot(h, w2_ref[...], preferred_element_type=jnp.float32)
    h2 = jnp.maximum(h2 + b2_ref[...], 0.0)
    mlp = jnp.sum(h2, axis=1, keepdims=True)
    o_ref[...] = jax.nn.sigmoid(lin + fm + mlp)


def _tc_deepfm(e, lr, xm, bias, w1, b1, w2, b2, block_b=4096):
    b, fd = e.shape
    f = xm.shape[1]
    h1, h2 = w1.shape[1], w2.shape[1]
    return pl.pallas_call(
        _tc_body,
        grid=(b // block_b,),
        in_specs=[
            pl.BlockSpec((block_b, fd), lambda i: (i, 0)),
            pl.BlockSpec((block_b, f * 16), lambda i: (i, 0)),
            pl.BlockSpec((block_b, f), lambda i: (i, 0)),
            pl.BlockSpec((1, 1), lambda i: (0, 0)),
            pl.BlockSpec((fd, h1), lambda i: (0, 0)),
            pl.BlockSpec((1, h1), lambda i: (0, 0)),
            pl.BlockSpec((h1, h2), lambda i: (0, 0)),
            pl.BlockSpec((1, h2), lambda i: (0, 0)),
        ],
        out_specs=pl.BlockSpec((block_b, 1), lambda i: (i, 0)),
        out_shape=jax.ShapeDtypeStruct((b, 1), jnp.float32),
    )(e, lr, xm, bias, w1, b1, w2, b2)


def kernel(x, emb_table, lin_table, bias, W1, b1, W2, b2):
    b, f = x.shape
    v, d = emb_table.shape
    xi = x.astype(jnp.int32)
    idx = xi.reshape(1, b * f)
    idx16 = idx >> 4
    xm = (xi & 15).astype(jnp.float32)
    lin2d = lin_table.reshape(v // 16, 16)
    emb_perm = _tc_relayout(emb_table.T)
    emb_lin = emb_perm.reshape(emb_perm.shape[0] * 8, d)
    kperm = ((idx >> 10) << 10) + ((idx & 127) << 3) + ((idx >> 7) & 7)
    emb_g, lin_g = _sc_gather(emb_lin, lin2d, kperm, idx16)
    e = emb_g.reshape(b, f * d)
    lr = lin_g.reshape(b, f * 16)
    out = _tc_deepfm(
        e,
        lr,
        xm,
        bias.reshape(1, 1),
        W1.astype(jnp.bfloat16),
        b1.reshape(1, -1),
        W2.astype(jnp.bfloat16),
        b2.reshape(1, -1),
    )
    return out.reshape(b)


# speedup vs baseline: 2.0593x; 1.0066x over previous
import functools

import jax
import jax.numpy as jnp
from jax import lax
from jax.experimental import pallas as pl
from jax.experimental.pallas import tpu as pltpu
from jax.experimental.pallas import tpu_sc as plsc

_WINDOW = 128


def _sc_gather(emb_table, lin2d, idx, idx16):
    n = idx.shape[1]
    d = emb_table.shape[1]
    mesh = plsc.VectorSubcoreMesh(core_axis_name="c", subcore_axis_name="s")

    @functools.partial(
        pl.kernel,
        out_type=[
            jax.ShapeDtypeStruct((n, d), emb_table.dtype),
            jax.ShapeDtypeStruct((n, 16), lin2d.dtype),
        ],
        mesh=mesh,
        compiler_params=pltpu.CompilerParams(use_tc_tiling_on_sc=False),
    )
    def gather_kernel(emb_hbm, lin_hbm, i_hbm, i16_hbm, emb_out, lin_out):
        def body(i_vmem, i16_vmem, emb_vmem, lin_vmem):
            pltpu.sync_copy(emb_hbm.at[i_vmem.at[0]], emb_vmem)
            pltpu.sync_copy(lin_hbm.at[i16_vmem.at[0]], lin_vmem)

        pltpu.emit_pipeline(
            body,
            grid=(n // _WINDOW,),
            in_specs=[
                pl.BlockSpec((1, _WINDOW), lambda i: (0, i)),
                pl.BlockSpec((1, _WINDOW), lambda i: (0, i)),
            ],
            out_specs=[
                pl.BlockSpec((_WINDOW, d), lambda i: (i, 0)),
                pl.BlockSpec((_WINDOW, 16), lambda i: (i, 0)),
            ],
            core_axis_name=("c", "s"),
            dimension_semantics=(pltpu.PARALLEL,),
        )(i_hbm, i16_hbm, emb_out, lin_out)

    return gather_kernel(emb_table, lin2d, idx, idx16)


def _transpose_body(xt_ref, o_ref):
    xt = xt_ref[...]
    d, vb = xt.shape
    ng = vb // 1024
    x4 = xt.reshape(d, ng, 8, 128)
    outs = []
    for g in range(ng):
        bg = x4[:, g].transpose((1, 0, 2)).reshape(128, 128)
        outs.append(bg.T)
    o_ref[...] = jnp.concatenate(outs, axis=0)


def _tc_relayout(emb_t, block_v=131072):
    d, v = emb_t.shape
    grid = (v + block_v - 1) // block_v
    return pl.pallas_call(
        _transpose_body,
        grid=(grid,),
        in_specs=[pl.BlockSpec((d, block_v), lambda i: (0, i))],
        out_specs=pl.BlockSpec((block_v // 8, 8 * d), lambda i: (i, 0)),
        out_shape=jax.ShapeDtypeStruct((grid * block_v // 8, 8 * d), emb_t.dtype),
    )(emb_t)


def _tc_body(e_ref, lr_ref, xm_ref, bias_ref, w1_ref, b1_ref, w2_ref, b2_ref,
             o_ref):
    e = e_ref[...]
    fd = w1_ref.shape[0]
    f = xm_ref.shape[1]
    d = fd // f
    sel = (
        lax.broadcasted_iota(jnp.int32, (fd, d), 0) % d
        == lax.broadcasted_iota(jnp.int32, (fd, d), 1)
    ).astype(jnp.float32)
    s = jnp.dot(e, sel, preferred_element_type=jnp.float32)
    fm = 0.5 * (
        jnp.sum(s * s, axis=1, keepdims=True)
        - jnp.sum(e * e, axis=1, keepdims=True)
    )
    rep_sel = (
        lax.broadcasted_iota(jnp.int32, (f, fd), 0)
        == lax.broadcasted_iota(jnp.int32, (f, fd), 1) // d
    ).astype(jnp.float32)
    rep = jnp.dot(xm_ref[...], rep_sel, preferred_element_type=jnp.float32)
    jpat = (lax.broadcasted_iota(jnp.int32, (1, fd), 1) % d).astype(jnp.float32)
    mask = (rep == jpat).astype(jnp.float32)
    lin = jnp.sum(lr_ref[...] * mask, axis=1, keepdims=True) + bias_ref[0, 0]
    e16 = e.astype(jnp.bfloat16)
    h = jnp.dot(e16, w1_ref[...], preferred_element_type=jnp.float32)
    h = jnp.maximum(h + b1_ref[...], 0.0).astype(jnp.bfloat16)
    h2 = jnp.dot(h, w2_ref[...], preferred_element_type=jnp.float32)
    h2 = jnp.maximum(h2 + b2_ref[...], 0.0)
    mlp = jnp.sum(h2, axis=1, keepdims=True)
    o_ref[...] = jax.nn.sigmoid(lin + fm + mlp)


def _tc_deepfm(e, lr, xm, bias, w1, b1, w2, b2, block_b=4096):
    b, fd = e.shape
    f = xm.shape[1]
    h1, h2 = w1.shape[1], w2.shape[1]
    return pl.pallas_call(
        _tc_body,
        grid=(b // block_b,),
        in_specs=[
            pl.BlockSpec((block_b, fd), lambda i: (i, 0)),
            pl.BlockSpec((block_b, f * 16), lambda i: (i, 0)),
            pl.BlockSpec((block_b, f), lambda i: (i, 0)),
            pl.BlockSpec((1, 1), lambda i: (0, 0)),
            pl.BlockSpec((fd, h1), lambda i: (0, 0)),
            pl.BlockSpec((1, h1), lambda i: (0, 0)),
            pl.BlockSpec((h1, h2), lambda i: (0, 0)),
            pl.BlockSpec((1, h2), lambda i: (0, 0)),
        ],
        out_specs=pl.BlockSpec((block_b, 1), lambda i: (i, 0)),
        out_shape=jax.ShapeDtypeStruct((b, 1), jnp.float32),
    )(e, lr, xm, bias, w1, b1, w2, b2)


def kernel(x, emb_table, lin_table, bias, W1, b1, W2, b2):
    b, f = x.shape
    v, d = emb_table.shape
    xi = x.astype(jnp.int32)
    idx = xi.reshape(1, b * f)
    idx16 = idx >> 4
    xm = (xi & 15).astype(jnp.float32)
    lin2d = lin_table.reshape(v // 16, 16)
    emb_perm = _tc_relayout(emb_table.T)
    emb_lin = emb_perm.reshape(emb_perm.shape[0] * 8, d)
    kperm = ((idx >> 10) << 10) + ((idx & 127) << 3) + ((idx >> 7) & 7)
    emb_g, lin_g = _sc_gather(emb_lin, lin2d, kperm, idx16)
    e = emb_g.reshape(b, f * d)
    lr = lin_g.reshape(b, f * 16)
    out = _tc_deepfm(
        e,
        lr,
        xm,
        bias.reshape(1, 1),
        W1.astype(jnp.bfloat16),
        b1.reshape(1, -1),
        W2.astype(jnp.bfloat16),
        b2.reshape(1, -1),
    )
    return out.reshape(b)
